# baseline (speedup 1.0000x reference)
import functools
import jax, jax.numpy as jnp
from jax.experimental import pallas as pl
from jax.experimental.pallas import tpu as pltpu

K = 8192
DIM = 32
NPIX = 6272
KCH = 1024


def _conv(x, w, b, s, p):
    y = jax.lax.conv_general_dilated(x, w, (s, s), [(p, p), (p, p)], dimension_numbers=('NCHW', 'OIHW', 'NCHW'))
    return y + b[None, :, None, None]


def _convt(x, w, b):
    w2 = jnp.flip(w, (2, 3)).transpose(1, 0, 2, 3)
    y = jax.lax.conv_general_dilated(x, w2, (1, 1), [(2, 2), (2, 2)], lhs_dilation=(2, 2), dimension_numbers=('NCHW', 'OIHW', 'NCHW'))
    return y + b[None, :, None, None]


def _res(x, w1, b1, w2, b2):
    t = jax.nn.relu(x)
    t = _conv(t, w1, b1, 1, 1)
    t = jax.nn.relu(t)
    t = _conv(t, w2, b2, 1, 0)
    return x + t


def _vq_body(esq_ref, emb2_ref, zb_ref, zsq_ref, idx_ref, best_ref, bidx_ref):
    i = pl.program_id(0)
    cross2 = jax.lax.dot_general(
        emb2_ref[...], zb_ref[...], (((1,), (0,)), ((), ())),
        preferred_element_type=jnp.float32)
    a = esq_ref[...] + cross2
    d = a + zsq_ref[...]
    cmin = jnp.min(d, axis=0, keepdims=True)
    rows = jax.lax.broadcasted_iota(jnp.int32, d.shape, 0)
    loc = jnp.min(jnp.where(d == cmin, rows, jnp.int32(2 ** 30)), axis=0, keepdims=True)
    gidx = loc + i * KCH

    @pl.when(i == 0)
    def _():
        best_ref[...] = cmin
        bidx_ref[...] = gidx

    @pl.when(i > 0)
    def _():
        upd = cmin < best_ref[...]
        bidx_ref[...] = jnp.where(upd, gidx, bidx_ref[...])
        best_ref[...] = jnp.where(upd, cmin, best_ref[...])

    @pl.when(i == pl.num_programs(0) - 1)
    def _():
        idx_ref[...] = bidx_ref[...]


def _vq_argmin(ze, codebook):
    zb = ze.transpose(1, 0, 2, 3).reshape(DIM, NPIX).astype(jnp.bfloat16)
    zsq = jnp.sum(ze ** 2, axis=1).reshape(1, NPIX)
    esq = jnp.sum(codebook ** 2, axis=1).reshape(K, 1)
    emb2 = codebook.astype(jnp.bfloat16) * jnp.bfloat16(-2.0)
    idx = pl.pallas_call(
        _vq_body,
        grid=(K // KCH,),
        in_specs=[
            pl.BlockSpec((KCH, 1), lambda i: (i, 0)),
            pl.BlockSpec((KCH, DIM), lambda i: (i, 0)),
            pl.BlockSpec((DIM, NPIX), lambda i: (0, 0)),
            pl.BlockSpec((1, NPIX), lambda i: (0, 0)),
        ],
        out_specs=pl.BlockSpec((1, NPIX), lambda i: (0, 0)),
        out_shape=jax.ShapeDtypeStruct((1, NPIX), jnp.int32),
        scratch_shapes=[
            pltpu.VMEM((1, NPIX), jnp.float32),
            pltpu.VMEM((1, NPIX), jnp.int32),
        ],
    )(esq, emb2, zb, zsq)
    return idx.reshape(2, 56, 56)


def kernel(x, enc_w1, enc_b1, enc_w2, enc_b2, enc_w3, enc_b3, r1_w1, r1_b1, r1_w2, r1_b2, r2_w1, r2_b1, r2_w2, r2_b2, codebook, dec_w1, dec_b1, r3_w1, r3_b1, r3_w2, r3_b2, r4_w1, r4_b1, r4_w2, r4_b2, dect_w2, dect_b2, dect_w3, dect_b3):
    h = _conv(x, enc_w1, enc_b1, 2, 1)
    h = jax.nn.relu(h)
    h = _conv(h, enc_w2, enc_b2, 2, 1)
    h = jax.nn.relu(h)
    h = _conv(h, enc_w3, enc_b3, 1, 1)
    h = _res(h, r1_w1, r1_b1, r1_w2, r1_b2)
    ze = _res(h, r2_w1, r2_b1, r2_w2, r2_b2)

    nn_idx = _vq_argmin(ze, codebook)

    usage = jnp.zeros((K,), dtype=jnp.float32).at[nn_idx.reshape(-1)].add(1.0)
    total = usage.sum()
    usage = jnp.where(total > 0, usage / total, usage)
    zq = jnp.take(codebook, nn_idx, axis=0).transpose(0, 3, 1, 2)
    dec_in = zq
    h = _conv(dec_in, dec_w1, dec_b1, 1, 1)
    h = _res(h, r3_w1, r3_b1, r3_w2, r3_b2)
    h = _res(h, r4_w1, r4_b1, r4_w2, r4_b2)
    h = _convt(h, dect_w2, dect_b2)
    h = jax.nn.relu(h)
    x_hat = _convt(h, dect_w3, dect_b3)
    return (x_hat, ze, zq, usage)


# speedup vs baseline: 1.0729x; 1.0729x over previous
import functools
import jax, jax.numpy as jnp
from jax import lax
from jax.experimental import pallas as pl
from jax.experimental.pallas import tpu as pltpu
from jax.experimental.pallas import tpu_sc as plsc

K = 8192
DIM = 32
NPIX = 6272
KCH = 1024

NC = 2
NS = 16
NW = NC * NS
BPW = 208
NPAD = NW * BPW
KPW = K // NW


def _conv(x, w, b, s, p):
    y = jax.lax.conv_general_dilated(x, w, (s, s), [(p, p), (p, p)], dimension_numbers=('NCHW', 'OIHW', 'NCHW'))
    return y + b[None, :, None, None]


def _convt(x, w, b):
    w2 = jnp.flip(w, (2, 3)).transpose(1, 0, 2, 3)
    y = jax.lax.conv_general_dilated(x, w2, (1, 1), [(2, 2), (2, 2)], lhs_dilation=(2, 2), dimension_numbers=('NCHW', 'OIHW', 'NCHW'))
    return y + b[None, :, None, None]


def _res(x, w1, b1, w2, b2):
    t = jax.nn.relu(x)
    t = _conv(t, w1, b1, 1, 1)
    t = jax.nn.relu(t)
    t = _conv(t, w2, b2, 1, 0)
    return x + t


def _vq_body(esq_ref, emb2_ref, zb_ref, zsq_ref, idx_ref, best_ref, bidx_ref):
    i = pl.program_id(0)
    cross2 = jax.lax.dot_general(
        emb2_ref[...], zb_ref[...], (((1,), (0,)), ((), ())),
        preferred_element_type=jnp.float32)
    a = esq_ref[...] + cross2
    d = a + zsq_ref[...]
    cmin = jnp.min(d, axis=0, keepdims=True)
    rows = jax.lax.broadcasted_iota(jnp.int32, d.shape, 0)
    loc = jnp.min(jnp.where(d == cmin, rows, jnp.int32(2 ** 30)), axis=0, keepdims=True)
    gidx = loc + i * KCH

    @pl.when(i == 0)
    def _():
        best_ref[...] = cmin
        bidx_ref[...] = gidx

    @pl.when(i > 0)
    def _():
        upd = cmin < best_ref[...]
        bidx_ref[...] = jnp.where(upd, gidx, bidx_ref[...])
        best_ref[...] = jnp.where(upd, cmin, best_ref[...])

    @pl.when(i == pl.num_programs(0) - 1)
    def _():
        idx_ref[...] = bidx_ref[...]


def _vq_argmin(ze, codebook):
    zb = ze.transpose(1, 0, 2, 3).reshape(DIM, NPIX).astype(jnp.bfloat16)
    zsq = jnp.sum(ze ** 2, axis=1).reshape(1, NPIX)
    esq = jnp.sum(codebook ** 2, axis=1).reshape(K, 1)
    emb2 = codebook.astype(jnp.bfloat16) * jnp.bfloat16(-2.0)
    idx = pl.pallas_call(
        _vq_body,
        grid=(K // KCH,),
        in_specs=[
            pl.BlockSpec((KCH, 1), lambda i: (i, 0)),
            pl.BlockSpec((KCH, DIM), lambda i: (i, 0)),
            pl.BlockSpec((DIM, NPIX), lambda i: (0, 0)),
            pl.BlockSpec((1, NPIX), lambda i: (0, 0)),
        ],
        out_specs=pl.BlockSpec((1, NPIX), lambda i: (0, 0)),
        out_shape=jax.ShapeDtypeStruct((1, NPIX), jnp.int32),
        scratch_shapes=[
            pltpu.VMEM((1, NPIX), jnp.float32),
            pltpu.VMEM((1, NPIX), jnp.int32),
        ],
    )(esq, emb2, zb, zsq)
    return idx.reshape(NPIX)


HW16 = 16
HROWS = 8448
SPW = HROWS // NS
GCH = BPW // 2


def _sc_body(codebook_hbm, idxg_hbm, idxs_hbm, ones_hbm, zeros_hbm, zq_hbm, upart_hbm,
             idxg_v, idxs_v, rows_v, ones_v, hist_sp, sem):
    c = lax.axis_index("c")
    s = lax.axis_index("s")
    wid = s * NC + c
    pltpu.sync_copy(zeros_hbm.at[pl.ds(s * SPW, SPW)], hist_sp.at[pl.ds(s * SPW, SPW)])
    pltpu.sync_copy(idxg_hbm.at[wid], idxg_v)
    pltpu.sync_copy(idxs_hbm.at[wid], idxs_v)
    pltpu.sync_copy(ones_hbm, ones_v)
    pltpu.async_copy(codebook_hbm.at[idxg_v.at[0]], rows_v.at[pl.ds(0, GCH)], sem).wait()
    pltpu.async_copy(codebook_hbm.at[idxg_v.at[1]], rows_v.at[pl.ds(GCH, GCH)], sem).wait()
    pltpu.sync_copy(rows_v, zq_hbm.at[pl.ds(wid * BPW, BPW)])
    plsc.subcore_barrier()
    pltpu.sync_copy(ones_v, hist_sp.at[idxs_v.at[0]], add=True)
    pltpu.sync_copy(ones_v, hist_sp.at[idxs_v.at[1]], add=True)
    plsc.subcore_barrier()
    pltpu.sync_copy(hist_sp.at[pl.ds(s * SPW, SPW)], upart_hbm.at[c, pl.ds(s * SPW, SPW)])


@functools.partial(
    pl.kernel,
    mesh=plsc.VectorSubcoreMesh(core_axis_name="c", subcore_axis_name="s"),
    compiler_params=pltpu.CompilerParams(use_tc_tiling_on_sc=False),
    out_type=[
        jax.ShapeDtypeStruct((NPAD, DIM), jnp.float32),
        jax.ShapeDtypeStruct((NC, HROWS, HW16), jnp.float32),
    ],
    scratch_types=[
        pltpu.VMEM((2, GCH), jnp.int32),
        pltpu.VMEM((2, GCH), jnp.int32),
        pltpu.VMEM((BPW, DIM), jnp.float32),
        pltpu.VMEM((GCH, HW16), jnp.float32),
        pltpu.VMEM_SHARED((HROWS, HW16), jnp.float32),
        pltpu.SemaphoreType.DMA,
    ],
)
def _sc_gather_usage(codebook_hbm, idxg_hbm, idxs_hbm, ones_hbm, zeros_hbm, zq_hbm, upart_hbm,
                     idxg_v, idxs_v, rows_v, ones_v, hist_sp, sem):
    _sc_body(codebook_hbm, idxg_hbm, idxs_hbm, ones_hbm, zeros_hbm, zq_hbm, upart_hbm,
             idxg_v, idxs_v, rows_v, ones_v, hist_sp, sem)


def kernel(x, enc_w1, enc_b1, enc_w2, enc_b2, enc_w3, enc_b3, r1_w1, r1_b1, r1_w2, r1_b2, r2_w1, r2_b1, r2_w2, r2_b2, codebook, dec_w1, dec_b1, r3_w1, r3_b1, r3_w2, r3_b2, r4_w1, r4_b1, r4_w2, r4_b2, dect_w2, dect_b2, dect_w3, dect_b3):
    h = _conv(x, enc_w1, enc_b1, 2, 1)
    h = jax.nn.relu(h)
    h = _conv(h, enc_w2, enc_b2, 2, 1)
    h = jax.nn.relu(h)
    h = _conv(h, enc_w3, enc_b3, 1, 1)
    h = _res(h, r1_w1, r1_b1, r1_w2, r1_b2)
    ze = _res(h, r2_w1, r2_b1, r2_w2, r2_b2)

    idx_flat = _vq_argmin(ze, codebook)
    idxg = jnp.concatenate([idx_flat, jnp.zeros((NPAD - NPIX,), jnp.int32)]).reshape(NW, 2, GCH)
    idxs = jnp.concatenate([idx_flat, jnp.full((NPAD - NPIX,), K, jnp.int32)]).reshape(NW, 2, GCH)
    ones_in = jnp.ones((GCH, HW16), jnp.float32)
    zeros_in = jnp.zeros((HROWS, HW16), jnp.float32)

    zq_flat, upart = _sc_gather_usage(codebook, idxg, idxs, ones_in, zeros_in)
    usage = (upart[0, :K, 0] + upart[1, :K, 0]) / jnp.float32(NPIX)
    zq = zq_flat[:NPIX].reshape(2, 56, 56, DIM).transpose(0, 3, 1, 2)

    dec_in = zq
    h = _conv(dec_in, dec_w1, dec_b1, 1, 1)
    h = _res(h, r3_w1, r3_b1, r3_w2, r3_b2)
    h = _res(h, r4_w1, r4_b1, r4_w2, r4_b2)
    h = _convt(h, dect_w2, dect_b2)
    h = jax.nn.relu(h)
    x_hat = _convt(h, dect_w3, dect_b3)
    return (x_hat, ze, zq, usage)
